# baseline (speedup 1.0000x reference)
import functools

import jax
import jax.numpy as jnp
from jax.experimental import pallas as pl

H = 128
NH = 4
HD = H // NH
SCALE = HD ** (-0.5)


def _dot_t(x, w):
    return jax.lax.dot_general(x, w, (((1,), (1,)), ((), ())),
                               preferred_element_type=jnp.float32)


def _dot_t_bf(x, w):
    return _dot_t(x, w).astype(jnp.bfloat16)


def _fused_kernel(zg_ref, zc_ref, zm_ref, wq_ref, wk_ref, wv_ref, wo_ref,
                  bo_ref, gamma_ref, beta_ref, ml_ref, out_ref):
    zg = zg_ref[...]
    bf = jnp.bfloat16
    zgb = zg.astype(bf)
    zcb = zc_ref[...].astype(bf)
    zmb = zm_ref[...].astype(bf)
    wq = wq_ref[...].astype(bf)
    wk = wk_ref[...].astype(bf)
    wv = wv_ref[...].astype(bf)

    q0 = _dot_t_bf(zgb, wq)
    kg = _dot_t_bf(zgb, wk)
    kc = _dot_t_bf(zcb, wk)
    km = _dot_t_bf(zmb, wk)
    vg = _dot_t(zgb, wv)
    vc = _dot_t(zcb, wv)
    vm = _dot_t(zmb, wv)

    li = jax.lax.broadcasted_iota(jnp.int32, (H, 12), 0)
    lj = jax.lax.broadcasted_iota(jnp.int32, (H, 12), 1)

    def seg(mod):
        return jnp.where(lj == mod * NH + li // HD, SCALE * 0.5, 0.0).astype(bf)

    def nrw(p, s):
        return jax.lax.dot_general(p, s, (((1,), (0,)), ((), ())),
                                   preferred_element_type=jnp.float32)

    s12 = (nrw(q0 * kg, seg(0)) + nrw(q0 * kc, seg(1))) + nrw(q0 * km, seg(2))
    st = jnp.transpose(s12, (1, 0))
    a = st[0:NH, :]
    b = st[NH:2 * NH, :]
    c = st[2 * NH:3 * NH, :]

    m = jnp.maximum(a, jnp.maximum(b, c))
    a = a - m
    b = b - m
    c = c - m
    x3 = jnp.minimum(a, jnp.minimum(b, c))
    x2 = (a + b + c) - x3
    t2 = x2 * x2
    tau2 = 0.5 * x2 - jnp.sqrt(jnp.maximum(0.5 - 0.25 * t2, 0.0))
    s3 = x2 + x3
    ss3 = (t2 + x3 * x3) - s3 * s3 * (1.0 / 3.0)
    tau3 = s3 * (1.0 / 3.0) - jnp.sqrt(jnp.maximum((1.0 - ss3) * (1.0 / 3.0), 0.0))
    c2 = tau2 <= x2
    c3 = tau3 <= x3
    tau_star = jnp.where(c2 & c3, tau3, jnp.where(c2 | c3, tau2, -1.0))

    def wgt(x):
        y = jnp.maximum(x - tau_star, 0.0)
        return y * y

    yt = jnp.concatenate([wgt(a), wgt(b), wgt(c)], axis=0).astype(bf)
    y12 = jnp.transpose(yt, (1, 0))

    def bsel(mod):
        ri = jax.lax.broadcasted_iota(jnp.int32, (12, H), 0)
        ci = jax.lax.broadcasted_iota(jnp.int32, (12, H), 1)
        return jnp.where(ri == mod * NH + ci // HD, 1.0, 0.0).astype(bf)

    def bcast(mod):
        return jax.lax.dot_general(y12, bsel(mod), (((1,), (0,)), ((), ())),
                                   preferred_element_type=jnp.float32)

    ctx = bcast(0) * vg + bcast(1) * vc + bcast(2) * vm

    ml = ml_ref[...]
    e = jnp.exp(ml - jnp.max(ml))
    w0 = e[0, 0] / jnp.sum(e)

    out = _dot_t(ctx.astype(bf), (wo_ref[...] * w0).astype(bf)) + bo_ref[...]
    t = out + zg

    jmat = jnp.full((H, H), 1.0 / H, jnp.float32)

    def lane_mean(x):
        return jax.lax.dot_general(x, jmat, (((1,), (0,)), ((), ())),
                                   preferred_element_type=jnp.float32)

    mu = lane_mean(t)
    var = lane_mean(t * t) - mu * mu
    dlt = t - mu
    out_ref[...] = dlt * jax.lax.rsqrt(var + 1e-5) * gamma_ref[...] + beta_ref[...]


@functools.partial(jax.jit, static_argnames=())
def kernel(z_gene, z_cpg, z_mirna, W_q, W_k, W_v, W_out, b_out, gamma, beta,
           modality_logits):
    B = z_gene.shape[0]
    Bb = 2000 if B % 2000 == 0 else B
    grid = (B // Bb,)

    row_spec = pl.BlockSpec((Bb, H), lambda i: (i, 0))
    w_spec = pl.BlockSpec((H, H), lambda i: (0, 0))
    vec_spec = pl.BlockSpec((1, H), lambda i: (0, 0))
    ml_spec = pl.BlockSpec((1, 3), lambda i: (0, 0))

    return pl.pallas_call(
        _fused_kernel,
        grid=grid,
        in_specs=[row_spec, row_spec, row_spec, w_spec, w_spec, w_spec,
                  w_spec, vec_spec, vec_spec, vec_spec, ml_spec],
        out_specs=row_spec,
        out_shape=jax.ShapeDtypeStruct((B, H), jnp.float32),
    )(z_gene, z_cpg, z_mirna, W_q, W_k, W_v, W_out,
      b_out.reshape(1, H), gamma.reshape(1, H), beta.reshape(1, H),
      modality_logits.reshape(1, 3))


# speedup vs baseline: 1.0829x; 1.0829x over previous
import functools

import jax
import jax.numpy as jnp
from jax.experimental import pallas as pl

H = 128
NH = 4
HD = H // NH
SCALE = HD ** (-0.5)


def _dot_t(x, w):
    return jax.lax.dot_general(x, w, (((1,), (1,)), ((), ())),
                               preferred_element_type=jnp.float32)


def _fused_kernel(zg_ref, zc_ref, zm_ref, wq_ref, wk_ref, wv_ref, wo_ref,
                  bo_ref, gamma_ref, beta_ref, ml_ref, out_ref):
    zg = zg_ref[...]
    zc = zc_ref[...]
    zm = zm_ref[...]
    wq = wq_ref[...]
    wk = wk_ref[...]
    wv = wv_ref[...]

    q0 = _dot_t(zg, wq)
    kg = _dot_t(zg, wk)
    kc = _dot_t(zc, wk)
    km = _dot_t(zm, wk)
    vg = _dot_t(zg, wv)
    vc = _dot_t(zc, wv)
    vm = _dot_t(zm, wv)

    li = jax.lax.broadcasted_iota(jnp.int32, (H, 12), 0)
    lj = jax.lax.broadcasted_iota(jnp.int32, (H, 12), 1)

    def seg(mod):
        return jnp.where(lj == mod * NH + li // HD, SCALE * 0.5, 0.0)

    def nrw(p, s):
        return jax.lax.dot_general(p, s, (((1,), (0,)), ((), ())),
                                   preferred_element_type=jnp.float32)

    s12 = (nrw(q0 * kg, seg(0)) + nrw(q0 * kc, seg(1))) + nrw(q0 * km, seg(2))
    st = jnp.transpose(s12, (1, 0))
    a = st[0:NH, :]
    b = st[NH:2 * NH, :]
    c = st[2 * NH:3 * NH, :]

    m = jnp.maximum(a, jnp.maximum(b, c))
    a = a - m
    b = b - m
    c = c - m
    x3 = jnp.minimum(a, jnp.minimum(b, c))
    x2 = (a + b + c) - x3
    t2 = x2 * x2
    tau2 = 0.5 * x2 - jnp.sqrt(jnp.maximum(0.5 - 0.25 * t2, 0.0))
    s3 = x2 + x3
    ss3 = (t2 + x3 * x3) - s3 * s3 * (1.0 / 3.0)
    tau3 = s3 * (1.0 / 3.0) - jnp.sqrt(jnp.maximum((1.0 - ss3) * (1.0 / 3.0), 0.0))
    c2 = tau2 <= x2
    c3 = tau3 <= x3
    tau_star = jnp.where(c2 & c3, tau3, jnp.where(c2 | c3, tau2, -1.0))

    def wgt(x):
        y = jnp.maximum(x - tau_star, 0.0)
        return y * y

    yt = jnp.concatenate([wgt(a), wgt(b), wgt(c)], axis=0)
    y12 = jnp.transpose(yt, (1, 0))

    def bsel(mod):
        ri = jax.lax.broadcasted_iota(jnp.int32, (12, H), 0)
        ci = jax.lax.broadcasted_iota(jnp.int32, (12, H), 1)
        return jnp.where(ri == mod * NH + ci // HD, 1.0, 0.0)

    def bcast(mod):
        return jax.lax.dot_general(y12, bsel(mod), (((1,), (0,)), ((), ())),
                                   preferred_element_type=jnp.float32)

    ctx = bcast(0) * vg + bcast(1) * vc + bcast(2) * vm

    ml = ml_ref[...]
    e = jnp.exp(ml - jnp.max(ml))
    w0 = e[0, 0] / jnp.sum(e)

    out = _dot_t(ctx, wo_ref[...] * w0) + bo_ref[...]
    t = out + zg

    jmat = jnp.full((H, H), 1.0 / H, jnp.float32)

    def lane_mean(x):
        return jax.lax.dot_general(x, jmat, (((1,), (0,)), ((), ())),
                                   preferred_element_type=jnp.float32)

    mu = lane_mean(t)
    var = lane_mean(t * t) - mu * mu
    dlt = t - mu
    out_ref[...] = dlt * jax.lax.rsqrt(var + 1e-5) * gamma_ref[...] + beta_ref[...]


@functools.partial(jax.jit, static_argnames=())
def kernel(z_gene, z_cpg, z_mirna, W_q, W_k, W_v, W_out, b_out, gamma, beta,
           modality_logits):
    B = z_gene.shape[0]
    Bb = 5000 if B % 5000 == 0 else B
    grid = (B // Bb,)

    row_spec = pl.BlockSpec((Bb, H), lambda i: (i, 0))
    w_spec = pl.BlockSpec((H, H), lambda i: (0, 0))
    vec_spec = pl.BlockSpec((1, H), lambda i: (0, 0))
    ml_spec = pl.BlockSpec((1, 3), lambda i: (0, 0))

    return pl.pallas_call(
        _fused_kernel,
        grid=grid,
        in_specs=[row_spec, row_spec, row_spec, w_spec, w_spec, w_spec,
                  w_spec, vec_spec, vec_spec, vec_spec, ml_spec],
        out_specs=row_spec,
        out_shape=jax.ShapeDtypeStruct((B, H), jnp.float32),
    )(z_gene, z_cpg, z_mirna, W_q, W_k, W_v, W_out,
      b_out.reshape(1, H), gamma.reshape(1, H), beta.reshape(1, H),
      modality_logits.reshape(1, 3))


# speedup vs baseline: 1.7689x; 1.6334x over previous
import functools

import jax
import jax.numpy as jnp
from jax.experimental import pallas as pl

H = 128
NH = 4
HD = H // NH
SCALE = HD ** (-0.5)


def _dot_t(x, w):
    return jax.lax.dot_general(x, w, (((1,), (1,)), ((), ())),
                               preferred_element_type=jnp.float32)


def _fused_kernel(zg_ref, zc_ref, zm_ref, wq_ref, wk_ref, wv_ref, wo_ref,
                  bo_ref, gamma_ref, beta_ref, ml_ref, out_ref):
    zg = zg_ref[...]
    bf = jnp.bfloat16
    zgb = zg.astype(bf)
    zcb = zc_ref[...].astype(bf)
    zmb = zm_ref[...].astype(bf)
    wq = wq_ref[...].astype(bf)
    wk = wk_ref[...].astype(bf)
    wv = wv_ref[...].astype(bf)

    q0 = _dot_t(zgb, wq)
    kg = _dot_t(zgb, wk)
    kc = _dot_t(zcb, wk)
    km = _dot_t(zmb, wk)
    vg = _dot_t(zgb, wv)
    vc = _dot_t(zcb, wv)
    vm = _dot_t(zmb, wv)

    li = jax.lax.broadcasted_iota(jnp.int32, (H, 12), 0)
    lj = jax.lax.broadcasted_iota(jnp.int32, (H, 12), 1)

    def seg(mod):
        return jnp.where(lj == mod * NH + li // HD, SCALE * 0.5, 0.0)

    def nrw(p, s):
        return jax.lax.dot_general(p, s, (((1,), (0,)), ((), ())),
                                   preferred_element_type=jnp.float32)

    s12 = (nrw(q0 * kg, seg(0)) + nrw(q0 * kc, seg(1))) + nrw(q0 * km, seg(2))
    st = jnp.transpose(s12, (1, 0))
    a = st[0:NH, :]
    b = st[NH:2 * NH, :]
    c = st[2 * NH:3 * NH, :]

    m = jnp.maximum(a, jnp.maximum(b, c))
    a = a - m
    b = b - m
    c = c - m
    x3 = jnp.minimum(a, jnp.minimum(b, c))
    x2 = (a + b + c) - x3
    t2 = x2 * x2
    tau2 = 0.5 * x2 - jnp.sqrt(jnp.maximum(0.5 - 0.25 * t2, 0.0))
    s3 = x2 + x3
    ss3 = (t2 + x3 * x3) - s3 * s3 * (1.0 / 3.0)
    tau3 = s3 * (1.0 / 3.0) - jnp.sqrt(jnp.maximum((1.0 - ss3) * (1.0 / 3.0), 0.0))
    c2 = tau2 <= x2
    c3 = tau3 <= x3
    tau_star = jnp.where(c2 & c3, tau3, jnp.where(c2 | c3, tau2, -1.0))

    def wgt(x):
        y = jnp.maximum(x - tau_star, 0.0)
        return y * y

    yt = jnp.concatenate([wgt(a), wgt(b), wgt(c)], axis=0)
    y12 = jnp.transpose(yt, (1, 0))

    def bsel(mod):
        ri = jax.lax.broadcasted_iota(jnp.int32, (12, H), 0)
        ci = jax.lax.broadcasted_iota(jnp.int32, (12, H), 1)
        return jnp.where(ri == mod * NH + ci // HD, 1.0, 0.0)

    def bcast(mod):
        return jax.lax.dot_general(y12, bsel(mod), (((1,), (0,)), ((), ())),
                                   preferred_element_type=jnp.float32)

    ctx = bcast(0) * vg + bcast(1) * vc + bcast(2) * vm

    ml = ml_ref[...]
    e = jnp.exp(ml - jnp.max(ml))
    w0 = e[0, 0] / jnp.sum(e)

    out = _dot_t(ctx, wo_ref[...] * w0) + bo_ref[...]
    t = out + zg

    jmat = jnp.full((H, H), 1.0 / H, jnp.float32)

    def lane_mean(x):
        return jax.lax.dot_general(x, jmat, (((1,), (0,)), ((), ())),
                                   preferred_element_type=jnp.float32)

    mu = lane_mean(t)
    var = lane_mean(t * t) - mu * mu
    dlt = t - mu
    out_ref[...] = dlt * jax.lax.rsqrt(var + 1e-5) * gamma_ref[...] + beta_ref[...]


@functools.partial(jax.jit, static_argnames=())
def kernel(z_gene, z_cpg, z_mirna, W_q, W_k, W_v, W_out, b_out, gamma, beta,
           modality_logits):
    B = z_gene.shape[0]
    Bb = 4000 if B % 4000 == 0 else B
    grid = (B // Bb,)

    row_spec = pl.BlockSpec((Bb, H), lambda i: (i, 0))
    w_spec = pl.BlockSpec((H, H), lambda i: (0, 0))
    vec_spec = pl.BlockSpec((1, H), lambda i: (0, 0))
    ml_spec = pl.BlockSpec((1, 3), lambda i: (0, 0))

    return pl.pallas_call(
        _fused_kernel,
        grid=grid,
        in_specs=[row_spec, row_spec, row_spec, w_spec, w_spec, w_spec,
                  w_spec, vec_spec, vec_spec, vec_spec, ml_spec],
        out_specs=row_spec,
        out_shape=jax.ShapeDtypeStruct((B, H), jnp.float32),
    )(z_gene, z_cpg, z_mirna, W_q, W_k, W_v, W_out,
      b_out.reshape(1, H), gamma.reshape(1, H), beta.reshape(1, H),
      modality_logits.reshape(1, 3))


# speedup vs baseline: 1.7723x; 1.0019x over previous
import functools

import jax
import jax.numpy as jnp
from jax.experimental import pallas as pl

H = 128
NH = 4
HD = H // NH
SCALE = HD ** (-0.5)


def _dot_t(x, w):
    return jax.lax.dot_general(x, w, (((1,), (1,)), ((), ())),
                               preferred_element_type=jnp.float32,
                               precision=jax.lax.Precision.DEFAULT)


def _fused_kernel(zg_ref, zc_ref, zm_ref, wq_ref, wk_ref, wv_ref, wo_ref,
                  bo_ref, gamma_ref, beta_ref, ml_ref, out_ref):
    zg = zg_ref[...]
    zc = zc_ref[...]
    zm = zm_ref[...]
    wq = wq_ref[...]
    wk = wk_ref[...]
    wv = wv_ref[...]

    q0 = _dot_t(zg, wq)
    kg = _dot_t(zg, wk)
    kc = _dot_t(zc, wk)
    km = _dot_t(zm, wk)
    vg = _dot_t(zg, wv)
    vc = _dot_t(zc, wv)
    vm = _dot_t(zm, wv)

    li = jax.lax.broadcasted_iota(jnp.int32, (H, 12), 0)
    lj = jax.lax.broadcasted_iota(jnp.int32, (H, 12), 1)

    def seg(mod):
        return jnp.where(lj == mod * NH + li // HD, SCALE * 0.5, 0.0)

    def nrw(p, s):
        return jax.lax.dot_general(p, s, (((1,), (0,)), ((), ())),
                                   preferred_element_type=jnp.float32)

    s12 = (nrw(q0 * kg, seg(0)) + nrw(q0 * kc, seg(1))) + nrw(q0 * km, seg(2))
    st = jnp.transpose(s12, (1, 0))
    a = st[0:NH, :]
    b = st[NH:2 * NH, :]
    c = st[2 * NH:3 * NH, :]

    m = jnp.maximum(a, jnp.maximum(b, c))
    a = a - m
    b = b - m
    c = c - m
    x3 = jnp.minimum(a, jnp.minimum(b, c))
    x2 = (a + b + c) - x3
    t2 = x2 * x2
    tau2 = 0.5 * x2 - jnp.sqrt(jnp.maximum(0.5 - 0.25 * t2, 0.0))
    s3 = x2 + x3
    ss3 = (t2 + x3 * x3) - s3 * s3 * (1.0 / 3.0)
    tau3 = s3 * (1.0 / 3.0) - jnp.sqrt(jnp.maximum((1.0 - ss3) * (1.0 / 3.0), 0.0))
    c2 = tau2 <= x2
    c3 = tau3 <= x3
    tau_star = jnp.where(c2 & c3, tau3, jnp.where(c2 | c3, tau2, -1.0))

    def wgt(x):
        y = jnp.maximum(x - tau_star, 0.0)
        return y * y

    yt = jnp.concatenate([wgt(a), wgt(b), wgt(c)], axis=0)
    y12 = jnp.transpose(yt, (1, 0))

    def bcast(mod):
        ri = jax.lax.broadcasted_iota(jnp.int32, (12, H), 0)
        ci = jax.lax.broadcasted_iota(jnp.int32, (12, H), 1)
        bsel = jnp.where(ri == mod * NH + ci // HD, 1.0, 0.0)
        return jax.lax.dot_general(y12, bsel, (((1,), (0,)), ((), ())),
                                   preferred_element_type=jnp.float32)

    ctx = bcast(0) * vg + bcast(1) * vc + bcast(2) * vm

    ml = ml_ref[...]
    e = jnp.exp(ml - jnp.max(ml))
    w0 = e[0, 0] / jnp.sum(e)

    out = _dot_t(ctx, wo_ref[...] * w0) + bo_ref[...]
    t = out + zg

    jmat = jnp.full((H, H), 1.0 / H, jnp.float32)

    def lane_mean(x):
        return jax.lax.dot_general(x, jmat, (((1,), (0,)), ((), ())),
                                   preferred_element_type=jnp.float32)

    mu = lane_mean(t)
    var = lane_mean(t * t) - mu * mu
    dlt = t - mu
    out_ref[...] = dlt * jax.lax.rsqrt(var + 1e-5) * gamma_ref[...] + beta_ref[...]


@functools.partial(jax.jit, static_argnames=())
def kernel(z_gene, z_cpg, z_mirna, W_q, W_k, W_v, W_out, b_out, gamma, beta,
           modality_logits):
    B = z_gene.shape[0]
    Bb = 4000 if B % 4000 == 0 else B
    grid = (B // Bb,)

    row_spec = pl.BlockSpec((Bb, H), lambda i: (i, 0))
    w_spec = pl.BlockSpec((H, H), lambda i: (0, 0))
    vec_spec = pl.BlockSpec((1, H), lambda i: (0, 0))
    ml_spec = pl.BlockSpec((1, 3), lambda i: (0, 0))

    return pl.pallas_call(
        _fused_kernel,
        grid=grid,
        in_specs=[row_spec, row_spec, row_spec, w_spec, w_spec, w_spec,
                  w_spec, vec_spec, vec_spec, vec_spec, ml_spec],
        out_specs=row_spec,
        out_shape=jax.ShapeDtypeStruct((B, H), jnp.float32),
    )(z_gene, z_cpg, z_mirna, W_q, W_k, W_v, W_out,
      b_out.reshape(1, H), gamma.reshape(1, H), beta.reshape(1, H),
      modality_logits.reshape(1, 3))


# speedup vs baseline: 1.7786x; 1.0036x over previous
import functools

import jax
import jax.numpy as jnp
from jax.experimental import pallas as pl
from jax.experimental.pallas import tpu as pltpu

H = 128
NH = 4
HD = H // NH
SCALE = HD ** (-0.5)


def _dot_t(x, w):
    return jax.lax.dot_general(x, w, (((1,), (1,)), ((), ())),
                               preferred_element_type=jnp.float32,
                               precision=jax.lax.Precision.DEFAULT)


def _fused_kernel(zg_ref, zc_ref, zm_ref, wq_ref, wk_ref, wv_ref, wo_ref,
                  bo_ref, gamma_ref, beta_ref, ml_ref, out_ref):
    zg = zg_ref[...]
    zc = zc_ref[...]
    zm = zm_ref[...]
    wq = wq_ref[...]
    wk = wk_ref[...]
    wv = wv_ref[...]

    q0 = _dot_t(zg, wq)
    kg = _dot_t(zg, wk)
    kc = _dot_t(zc, wk)
    km = _dot_t(zm, wk)
    vg = _dot_t(zg, wv)
    vc = _dot_t(zc, wv)
    vm = _dot_t(zm, wv)

    li = jax.lax.broadcasted_iota(jnp.int32, (H, 12), 0)
    lj = jax.lax.broadcasted_iota(jnp.int32, (H, 12), 1)

    def seg(mod):
        return jnp.where(lj == mod * NH + li // HD, SCALE * 0.5, 0.0)

    def nrw(p, s):
        return jax.lax.dot_general(p, s, (((1,), (0,)), ((), ())),
                                   preferred_element_type=jnp.float32)

    s12 = (nrw(q0 * kg, seg(0)) + nrw(q0 * kc, seg(1))) + nrw(q0 * km, seg(2))
    st = jnp.transpose(s12, (1, 0))
    a = st[0:NH, :]
    b = st[NH:2 * NH, :]
    c = st[2 * NH:3 * NH, :]

    m = jnp.maximum(a, jnp.maximum(b, c))
    a = a - m
    b = b - m
    c = c - m
    x3 = jnp.minimum(a, jnp.minimum(b, c))
    x2 = (a + b + c) - x3
    t2 = x2 * x2
    tau2 = 0.5 * x2 - jnp.sqrt(jnp.maximum(0.5 - 0.25 * t2, 0.0))
    s3 = x2 + x3
    ss3 = (t2 + x3 * x3) - s3 * s3 * (1.0 / 3.0)
    tau3 = s3 * (1.0 / 3.0) - jnp.sqrt(jnp.maximum((1.0 - ss3) * (1.0 / 3.0), 0.0))
    c2 = tau2 <= x2
    c3 = tau3 <= x3
    tau_star = jnp.where(c2 & c3, tau3, jnp.where(c2 | c3, tau2, -1.0))

    def wgt(x):
        y = jnp.maximum(x - tau_star, 0.0)
        return y * y

    yt = jnp.concatenate([wgt(a), wgt(b), wgt(c)], axis=0)
    y12 = jnp.transpose(yt, (1, 0))

    def bcast(mod):
        ri = jax.lax.broadcasted_iota(jnp.int32, (12, H), 0)
        ci = jax.lax.broadcasted_iota(jnp.int32, (12, H), 1)
        bsel = jnp.where(ri == mod * NH + ci // HD, 1.0, 0.0)
        return jax.lax.dot_general(y12, bsel, (((1,), (0,)), ((), ())),
                                   preferred_element_type=jnp.float32)

    ctx = bcast(0) * vg + bcast(1) * vc + bcast(2) * vm

    ml = ml_ref[...]
    e = jnp.exp(ml - jnp.max(ml))
    w0 = e[0, 0] / jnp.sum(e)

    out = _dot_t(ctx, wo_ref[...] * w0) + bo_ref[...]
    t = out + zg

    jmat = jnp.full((H, H), 1.0 / H, jnp.float32)

    def lane_mean(x):
        return jax.lax.dot_general(x, jmat, (((1,), (0,)), ((), ())),
                                   preferred_element_type=jnp.float32)

    mu = lane_mean(t)
    var = lane_mean(t * t) - mu * mu
    dlt = t - mu
    out_ref[...] = dlt * jax.lax.rsqrt(var + 1e-5) * gamma_ref[...] + beta_ref[...]


@functools.partial(jax.jit, static_argnames=())
def kernel(z_gene, z_cpg, z_mirna, W_q, W_k, W_v, W_out, b_out, gamma, beta,
           modality_logits):
    B = z_gene.shape[0]
    Bb = 4000 if B % 4000 == 0 else B
    grid = (B // Bb,)

    row_spec = pl.BlockSpec((Bb, H), lambda i: (i, 0))
    w_spec = pl.BlockSpec((H, H), lambda i: (0, 0))
    vec_spec = pl.BlockSpec((1, H), lambda i: (0, 0))
    ml_spec = pl.BlockSpec((1, 3), lambda i: (0, 0))

    return pl.pallas_call(
        _fused_kernel,
        grid=grid,
        in_specs=[row_spec, row_spec, row_spec, w_spec, w_spec, w_spec,
                  w_spec, vec_spec, vec_spec, vec_spec, ml_spec],
        out_specs=row_spec,
        out_shape=jax.ShapeDtypeStruct((B, H), jnp.float32),
        compiler_params=pltpu.CompilerParams(
            dimension_semantics=("parallel",)),
    )(z_gene, z_cpg, z_mirna, W_q, W_k, W_v, W_out,
      b_out.reshape(1, H), gamma.reshape(1, H), beta.reshape(1, H),
      modality_logits.reshape(1, 3))
